# baseline (speedup 1.0000x reference)
import functools
import math

import jax
import jax.numpy as jnp
from jax import lax
from jax.experimental import pallas as pl
from jax.experimental.pallas import tpu as pltpu
from jax.experimental.pallas import tpu_sc as plsc

F32 = jnp.float32
_LOG2 = math.log(2.0)

N_ATOMS = 10000
N_INTER = 320000
N_IN = 64
N_FILT = 128

NC = 2
NS = 16
N_SLICES = 2
E_SLICE = N_INTER // N_SLICES
EDGES_PER_WORKER = E_SLICE // (NC * NS)
CHUNK = 40
N_CHUNKS = EDGES_PER_WORKER // CHUNK
PAD_ATOMS = 10240
STRIPE = PAD_ATOMS // NS


def _ssp(v):
    return jnp.maximum(v, 0.0) + jnp.log1p(jnp.exp(-jnp.abs(v))) - _LOG2


def _dot3(a, b):
    bf = jnp.bfloat16
    ah = a.astype(bf)
    al = (a - ah.astype(F32)).astype(bf)
    bh = b.astype(bf)
    bl = (b - bh.astype(F32)).astype(bf)
    hi = jnp.dot(ah, bh, preferred_element_type=F32)
    mid = (jnp.dot(ah, bl, preferred_element_type=F32)
           + jnp.dot(al, bh, preferred_element_type=F32))
    return hi + mid


def _filter_body(d_ref, w1_ref, b1_ref, w2_ref, b2_ref, o_ref):
    h = _ssp(_dot3(d_ref[...], w1_ref[...]) + b1_ref[...])
    w = _dot3(h, w2_ref[...])
    o_ref[...] = _ssp(w + b2_ref[...])


def _proj_body(x_ref, w_ref, o_ref):
    o_ref[...] = _dot3(x_ref[...], w_ref[...])


def _final_body(*refs):
    conv_refs = refs[:N_SLICES]
    x_ref, wf_ref, bf_ref, wd_ref, bd_ref, y_ref, v_ref = refs[N_SLICES:]
    conv = conv_refs[0][0] + conv_refs[0][1]
    for r in conv_refs[1:]:
        conv = conv + r[0] + r[1]
    c = _ssp(_dot3(conv, wf_ref[...]) + bf_ref[...])
    v = _dot3(c, wd_ref[...]) + bd_ref[...]
    v_ref[...] = v
    y_ref[...] = x_ref[...] + v


def _full(shape):
    return pl.BlockSpec(shape, lambda i: (0,) * len(shape))


def _rows(block_rows, ncols):
    return pl.BlockSpec((block_rows, ncols), lambda i: (i, 0))


LAST = N_CHUNKS - 1


def _sc_body(w_hbm, f_hbm, idxj_hbm, segi_hbm, segj_hbm, out_hbm,
             sj_v0, sj_v1, g_v0, g_v1, i2_0, i2_1, i2_2, i2_3,
             w_v0, w_v1, f_v0, f_v1, p_v0, p_v1,
             acc_sh, sem_sj, sem_g, sem_i, sem_f, sem_w, sem_sc):
    c = lax.axis_index("c")
    s = lax.axis_index("s")
    sj_v = (sj_v0, sj_v1)
    g_v = (g_v0, g_v1)
    i2_v = (i2_0, i2_1, i2_2, i2_3)
    w_v = (w_v0, w_v1)
    f_v = (f_v0, f_v1)
    p_v = (p_v0, p_v1)

    def _zrow(r, carry):
        for cg in range(N_FILT // 16):
            f_v0[r, pl.ds(cg * 16, 16)] = jnp.zeros((16,), F32)
        return carry

    lax.fori_loop(0, CHUNK, _zrow, 0)
    for j in range(STRIPE // CHUNK):
        pltpu.sync_copy(f_v0, acc_sh.at[pl.ds(s * STRIPE + j * CHUNK, CHUNK)])
    plsc.subcore_barrier()

    base0 = (c * NS + s) * EDGES_PER_WORKER

    def _base(t):
        return pl.multiple_of(base0 + t * CHUNK, 8)

    def issue_sj(t, b):
        pltpu.async_copy(segj_hbm.at[pl.ds(_base(t), CHUNK)], sj_v[b],
                         sem_sj.at[b])

    def wait_sj(b):
        pltpu.make_async_copy(segj_hbm.at[pl.ds(0, CHUNK)], sj_v[b],
                              sem_sj.at[b]).wait()

    def issue_gi(b, q):
        pltpu.async_copy(idxj_hbm.at[sj_v[b]], g_v[b], sem_g.at[b])
        pltpu.async_copy(segi_hbm.at[sj_v[b]], i2_v[q], sem_i.at[b])

    def wait_gi(b, q):
        pltpu.make_async_copy(idxj_hbm.at[sj_v[b]], g_v[b], sem_g.at[b]).wait()
        pltpu.make_async_copy(segi_hbm.at[sj_v[b]], i2_v[q],
                              sem_i.at[b]).wait()

    def issue_dat(t, b):
        pltpu.async_copy(f_hbm.at[g_v[b]], f_v[b], sem_f.at[b])
        pltpu.async_copy(w_hbm.at[pl.ds(_base(t), CHUNK)], w_v[b],
                         sem_w.at[b])

    def wait_dat(b):
        pltpu.make_async_copy(f_hbm.at[g_v[b]], f_v[b], sem_f.at[b]).wait()
        pltpu.make_async_copy(w_hbm.at[pl.ds(0, CHUNK)], w_v[b],
                              sem_w.at[b]).wait()

    def wait_sc(b, q):
        pltpu.make_async_copy(p_v[b], acc_sh.at[i2_v[q]], sem_sc.at[b]).wait()

    def multiply(b):
        def _mul_row(r, inner):
            for cg in range(N_FILT // 16):
                sl = pl.ds(cg * 16, 16)
                p_v[b][r, sl] = w_v[b][r, sl] * f_v[b][r, sl]
            return inner

        lax.fori_loop(0, CHUNK, _mul_row, 0)

    def scatter(b, q):
        pltpu.async_copy(p_v[b], acc_sh.at[i2_v[q]], sem_sc.at[b], add=True)

    issue_sj(0, 0)
    wait_sj(0)
    issue_gi(0, 0)
    issue_sj(1, 1)
    wait_gi(0, 0)
    issue_dat(0, 0)
    wait_sj(1)
    issue_gi(1, 1)
    issue_sj(2, 0)

    def _outer(i, carry):
        for j in range(4):
            t = i * 4 + j
            b = j % 2
            o = 1 - b
            q = j
            wait_dat(b)
            wait_gi(o, (q + 1) % 4)
            issue_dat(t + 1, o)
            if j < 2:
                @pl.when(i > 0)
                def _():
                    wait_sc(b, (q + 2) % 4)
            else:
                wait_sc(b, (q + 2) % 4)
            if j == 3:
                @pl.when(t + 2 <= LAST)
                def _():
                    wait_sj(b)
                    issue_gi(b, (q + 2) % 4)
            else:
                wait_sj(b)
                issue_gi(b, (q + 2) % 4)
            if j >= 2:
                @pl.when(t + 3 <= LAST)
                def _():
                    issue_sj(t + 3, o)
            else:
                issue_sj(t + 3, o)
            multiply(b)
            scatter(b, q)
        return carry

    n_main = 4 * ((N_CHUNKS - 1) // 4)
    lax.fori_loop(0, (N_CHUNKS - 1) // 4, _outer, 0)

    for t in range(n_main, N_CHUNKS):
        b, q = t % 2, t % 4
        wait_dat(b)
        if t + 1 < N_CHUNKS:
            wait_gi((t + 1) % 2, (t + 1) % 4)
            issue_dat(t + 1, (t + 1) % 2)
        if t >= 2:
            wait_sc(b, (q + 2) % 4)
        multiply(b)
        scatter(b, q)
    wait_sc((N_CHUNKS - 2) % 2, (N_CHUNKS - 2) % 4)
    wait_sc((N_CHUNKS - 1) % 2, (N_CHUNKS - 1) % 4)
    plsc.subcore_barrier()

    for j in range(STRIPE // CHUNK):
        r0 = s * STRIPE + j * CHUNK
        pltpu.sync_copy(acc_sh.at[pl.ds(r0, CHUNK)],
                        out_hbm.at[c, pl.ds(r0, CHUNK)])


@functools.lru_cache(maxsize=1)
def _sc_conv():
    ivec = pltpu.VMEM((CHUNK,), jnp.int32)
    fbuf = pltpu.VMEM((CHUNK, N_FILT), F32)
    return pl.kernel(
        _sc_body,
        out_type=jax.ShapeDtypeStruct((NC, PAD_ATOMS, N_FILT), F32),
        mesh=plsc.VectorSubcoreMesh(core_axis_name="c", subcore_axis_name="s",
                                    num_cores=NC, num_subcores=NS),
        scratch_types=[
            ivec, ivec, ivec, ivec, ivec, ivec, ivec, ivec,
            fbuf, fbuf, fbuf, fbuf, fbuf, fbuf,
            pltpu.VMEM_SHARED((PAD_ATOMS, N_FILT), F32),
            pltpu.SemaphoreType.DMA((2,)),
            pltpu.SemaphoreType.DMA((2,)),
            pltpu.SemaphoreType.DMA((2,)),
            pltpu.SemaphoreType.DMA((2,)),
            pltpu.SemaphoreType.DMA((2,)),
            pltpu.SemaphoreType.DMA((2,)),
        ],
    )


def kernel(x, dijk, idx_j, seg_i, seg_j, seg_i_sum,
           W1, b1, W2, b2, W_in2fac, W_fac2out, b_fac2out, W_dense, b_dense):
    del seg_i_sum
    assert x.shape == (N_ATOMS, N_FILT) and dijk.shape == (N_INTER, N_IN)

    b1r = b1.reshape(1, N_FILT)
    b2r = b2.reshape(1, N_FILT)
    bfr = b_fac2out.reshape(1, N_FILT)
    bdr = b_dense.reshape(1, N_FILT)

    BA = 2000
    f = pl.pallas_call(
        _proj_body,
        grid=(N_ATOMS // BA,),
        in_specs=[_rows(BA, N_FILT), _full((N_FILT, N_FILT))],
        out_specs=_rows(BA, N_FILT),
        out_shape=jax.ShapeDtypeStruct((N_ATOMS, N_FILT), F32),
    )(x, W_in2fac)

    BR = 2000
    blocks_per_slice = E_SLICE // BR
    partials = []
    for sl in range(N_SLICES):
        lo = sl * E_SLICE
        sj_sl = lax.slice_in_dim(seg_j, lo, lo + E_SLICE, axis=0)
        d_spec = pl.BlockSpec(
            (BR, N_IN), lambda i, sl=sl: (sl * blocks_per_slice + i, 0))
        w_sl = pl.pallas_call(
            _filter_body,
            grid=(blocks_per_slice,),
            in_specs=[d_spec, _full((N_IN, N_FILT)),
                      _full((1, N_FILT)), _full((N_FILT, N_FILT)),
                      _full((1, N_FILT))],
            out_specs=_rows(BR, N_FILT),
            out_shape=jax.ShapeDtypeStruct((E_SLICE, N_FILT), F32),
        )(dijk, W1, b1r, W2, b2r)
        partials.append(_sc_conv()(w_sl, f, idx_j, seg_i, sj_sl))

    p_spec = pl.BlockSpec((NC, BA, N_FILT), lambda i: (0, i, 0))
    y, v = pl.pallas_call(
        _final_body,
        grid=(N_ATOMS // BA,),
        in_specs=([p_spec] * N_SLICES
                  + [_rows(BA, N_FILT),
                     _full((N_FILT, N_FILT)), _full((1, N_FILT)),
                     _full((N_FILT, N_FILT)), _full((1, N_FILT))]),
        out_specs=[_rows(BA, N_FILT), _rows(BA, N_FILT)],
        out_shape=[jax.ShapeDtypeStruct((N_ATOMS, N_FILT), F32),
                   jax.ShapeDtypeStruct((N_ATOMS, N_FILT), F32)],
    )(*partials, x, W_fac2out, bfr, W_dense, bdr)

    return (y, v)


# speedup vs baseline: 1.1734x; 1.1734x over previous
import functools
import math

import jax
import jax.numpy as jnp
from jax import lax
from jax.experimental import pallas as pl
from jax.experimental.pallas import tpu as pltpu
from jax.experimental.pallas import tpu_sc as plsc

F32 = jnp.float32
_LOG2 = math.log(2.0)

N_ATOMS = 10000
N_INTER = 320000
N_IN = 64
N_FILT = 128

NC = 2
NS = 16
N_SLICES = 5
E_SLICE = N_INTER // N_SLICES
EDGES_PER_WORKER = E_SLICE // (NC * NS)
CHUNK = 80
N_CHUNKS = EDGES_PER_WORKER // CHUNK
PAD_ATOMS = 10240
STRIPE = PAD_ATOMS // NS


def _ssp(v):
    return jnp.maximum(v, 0.0) + jnp.log1p(jnp.exp(-jnp.abs(v))) - _LOG2


def _dot3(a, b):
    bf = jnp.bfloat16
    ah = a.astype(bf)
    al = (a - ah.astype(F32)).astype(bf)
    bh = b.astype(bf)
    bl = (b - bh.astype(F32)).astype(bf)
    hi = jnp.dot(ah, bh, preferred_element_type=F32)
    mid = (jnp.dot(ah, bl, preferred_element_type=F32)
           + jnp.dot(al, bh, preferred_element_type=F32))
    return hi + mid


def _filter_body(d_ref, w1_ref, b1_ref, w2_ref, b2_ref, o_ref):
    h = _ssp(_dot3(d_ref[...], w1_ref[...]) + b1_ref[...])
    w = _dot3(h, w2_ref[...])
    o_ref[...] = _ssp(w + b2_ref[...])


def _proj_body(x_ref, w_ref, o_ref):
    o_ref[...] = _dot3(x_ref[...], w_ref[...])


def _final_body(*refs):
    conv_refs = refs[:N_SLICES]
    x_ref, wf_ref, bf_ref, wd_ref, bd_ref, y_ref, v_ref = refs[N_SLICES:]
    conv = conv_refs[0][0] + conv_refs[0][1]
    for r in conv_refs[1:]:
        conv = conv + r[0] + r[1]
    c = _ssp(_dot3(conv, wf_ref[...]) + bf_ref[...])
    v = _dot3(c, wd_ref[...]) + bd_ref[...]
    v_ref[...] = v
    y_ref[...] = x_ref[...] + v


def _full(shape):
    return pl.BlockSpec(shape, lambda i: (0,) * len(shape))


def _rows(block_rows, ncols):
    return pl.BlockSpec((block_rows, ncols), lambda i: (i, 0))


LAST = N_CHUNKS - 1


def _sc_body(w_hbm, f_hbm, idxj_hbm, segi_hbm, segj_hbm, out_hbm,
             sj_v0, sj_v1, g_v0, g_v1, i2_0, i2_1, i2_2, i2_3,
             w_v0, w_v1, f_v0, f_v1,
             acc_sh, sem_sj, sem_g, sem_i, sem_f, sem_w, sem_sc):
    c = lax.axis_index("c")
    s = lax.axis_index("s")
    sj_v = (sj_v0, sj_v1)
    g_v = (g_v0, g_v1)
    i2_v = (i2_0, i2_1, i2_2, i2_3)
    w_v = (w_v0, w_v1)
    f_v = (f_v0, f_v1)

    def _zrow(r, carry):
        for cg in range(N_FILT // 16):
            f_v0[r, pl.ds(cg * 16, 16)] = jnp.zeros((16,), F32)
        return carry

    lax.fori_loop(0, CHUNK, _zrow, 0)
    for j in range(STRIPE // CHUNK):
        pltpu.sync_copy(f_v0, acc_sh.at[pl.ds(s * STRIPE + j * CHUNK, CHUNK)])
    plsc.subcore_barrier()

    base0 = (c * NS + s) * EDGES_PER_WORKER

    def _base(t):
        return pl.multiple_of(base0 + t * CHUNK, 8)

    def issue_sj(t, b):
        pltpu.async_copy(segj_hbm.at[pl.ds(_base(t), CHUNK)], sj_v[b],
                         sem_sj.at[b])

    def wait_sj(b):
        pltpu.make_async_copy(segj_hbm.at[pl.ds(0, CHUNK)], sj_v[b],
                              sem_sj.at[b]).wait()

    def issue_gi(b, q):
        pltpu.async_copy(idxj_hbm.at[sj_v[b]], g_v[b], sem_g.at[b])
        pltpu.async_copy(segi_hbm.at[sj_v[b]], i2_v[q], sem_i.at[b])

    def wait_gi(b, q):
        pltpu.make_async_copy(idxj_hbm.at[sj_v[b]], g_v[b], sem_g.at[b]).wait()
        pltpu.make_async_copy(segi_hbm.at[sj_v[b]], i2_v[q],
                              sem_i.at[b]).wait()

    def issue_dat(t, b):
        pltpu.async_copy(f_hbm.at[g_v[b]], f_v[b], sem_f.at[b])
        pltpu.async_copy(w_hbm.at[pl.ds(_base(t), CHUNK)], w_v[b],
                         sem_w.at[b])

    def wait_dat(b):
        pltpu.make_async_copy(f_hbm.at[g_v[b]], f_v[b], sem_f.at[b]).wait()
        pltpu.make_async_copy(w_hbm.at[pl.ds(0, CHUNK)], w_v[b],
                              sem_w.at[b]).wait()

    def wait_sc(b, q):
        pltpu.make_async_copy(w_v[b], acc_sh.at[i2_v[q]], sem_sc.at[b]).wait()

    def multiply(b):
        def _mul_row(r, inner):
            for cg in range(N_FILT // 16):
                sl = pl.ds(cg * 16, 16)
                w_v[b][r, sl] = w_v[b][r, sl] * f_v[b][r, sl]
            return inner

        lax.fori_loop(0, CHUNK, _mul_row, 0)

    def scatter(b, q):
        pltpu.async_copy(w_v[b], acc_sh.at[i2_v[q]], sem_sc.at[b], add=True)

    issue_sj(0, 0)
    wait_sj(0)
    issue_gi(0, 0)
    issue_sj(1, 1)
    wait_gi(0, 0)
    issue_dat(0, 0)
    wait_sj(1)
    issue_gi(1, 1)
    issue_sj(2, 0)

    def _outer(i, carry):
        for j in range(4):
            t = i * 4 + j
            b = j % 2
            o = 1 - b
            q = j
            wait_dat(b)
            if j == 0:
                @pl.when(i > 0)
                def _():
                    wait_sc(o, (q + 3) % 4)
            else:
                wait_sc(o, (q + 3) % 4)
            wait_gi(o, (q + 1) % 4)
            issue_dat(t + 1, o)
            if j == 3:
                @pl.when(t + 2 <= LAST)
                def _():
                    wait_sj(b)
                    issue_gi(b, (q + 2) % 4)
            else:
                wait_sj(b)
                issue_gi(b, (q + 2) % 4)
            if j >= 2:
                @pl.when(t + 3 <= LAST)
                def _():
                    issue_sj(t + 3, o)
            else:
                issue_sj(t + 3, o)
            multiply(b)
            scatter(b, q)
        return carry

    n_main = 4 * ((N_CHUNKS - 1) // 4)
    lax.fori_loop(0, (N_CHUNKS - 1) // 4, _outer, 0)

    for t in range(n_main, N_CHUNKS):
        b, q = t % 2, t % 4
        wait_dat(b)
        if t >= 1:
            wait_sc((t - 1) % 2, (t - 1) % 4)
        if t + 1 < N_CHUNKS:
            wait_gi((t + 1) % 2, (t + 1) % 4)
            issue_dat(t + 1, (t + 1) % 2)
        multiply(b)
        scatter(b, q)
    wait_sc((N_CHUNKS - 1) % 2, (N_CHUNKS - 1) % 4)
    plsc.subcore_barrier()

    for j in range(STRIPE // CHUNK):
        r0 = s * STRIPE + j * CHUNK
        pltpu.sync_copy(acc_sh.at[pl.ds(r0, CHUNK)],
                        out_hbm.at[c, pl.ds(r0, CHUNK)])


@functools.lru_cache(maxsize=1)
def _sc_conv():
    ivec = pltpu.VMEM((CHUNK,), jnp.int32)
    fbuf = pltpu.VMEM((CHUNK, N_FILT), F32)
    return pl.kernel(
        _sc_body,
        out_type=jax.ShapeDtypeStruct((NC, PAD_ATOMS, N_FILT), F32),
        mesh=plsc.VectorSubcoreMesh(core_axis_name="c", subcore_axis_name="s",
                                    num_cores=NC, num_subcores=NS),
        scratch_types=[
            ivec, ivec, ivec, ivec, ivec, ivec, ivec, ivec,
            fbuf, fbuf, fbuf, fbuf,
            pltpu.VMEM_SHARED((PAD_ATOMS, N_FILT), F32),
            pltpu.SemaphoreType.DMA((2,)),
            pltpu.SemaphoreType.DMA((2,)),
            pltpu.SemaphoreType.DMA((2,)),
            pltpu.SemaphoreType.DMA((2,)),
            pltpu.SemaphoreType.DMA((2,)),
            pltpu.SemaphoreType.DMA((2,)),
        ],
    )


def kernel(x, dijk, idx_j, seg_i, seg_j, seg_i_sum,
           W1, b1, W2, b2, W_in2fac, W_fac2out, b_fac2out, W_dense, b_dense):
    del seg_i_sum
    assert x.shape == (N_ATOMS, N_FILT) and dijk.shape == (N_INTER, N_IN)

    b1r = b1.reshape(1, N_FILT)
    b2r = b2.reshape(1, N_FILT)
    bfr = b_fac2out.reshape(1, N_FILT)
    bdr = b_dense.reshape(1, N_FILT)

    BA = 2000
    f = pl.pallas_call(
        _proj_body,
        grid=(N_ATOMS // BA,),
        in_specs=[_rows(BA, N_FILT), _full((N_FILT, N_FILT))],
        out_specs=_rows(BA, N_FILT),
        out_shape=jax.ShapeDtypeStruct((N_ATOMS, N_FILT), F32),
    )(x, W_in2fac)

    BR = 2000
    blocks_per_slice = E_SLICE // BR
    partials = []
    for sl in range(N_SLICES):
        lo = sl * E_SLICE
        sj_sl = lax.slice_in_dim(seg_j, lo, lo + E_SLICE, axis=0)
        d_spec = pl.BlockSpec(
            (BR, N_IN), lambda i, sl=sl: (sl * blocks_per_slice + i, 0))
        w_sl = pl.pallas_call(
            _filter_body,
            grid=(blocks_per_slice,),
            in_specs=[d_spec, _full((N_IN, N_FILT)),
                      _full((1, N_FILT)), _full((N_FILT, N_FILT)),
                      _full((1, N_FILT))],
            out_specs=_rows(BR, N_FILT),
            out_shape=jax.ShapeDtypeStruct((E_SLICE, N_FILT), F32),
        )(dijk, W1, b1r, W2, b2r)
        partials.append(_sc_conv()(w_sl, f, idx_j, seg_i, sj_sl))

    p_spec = pl.BlockSpec((NC, BA, N_FILT), lambda i: (0, i, 0))
    y, v = pl.pallas_call(
        _final_body,
        grid=(N_ATOMS // BA,),
        in_specs=([p_spec] * N_SLICES
                  + [_rows(BA, N_FILT),
                     _full((N_FILT, N_FILT)), _full((1, N_FILT)),
                     _full((N_FILT, N_FILT)), _full((1, N_FILT))]),
        out_specs=[_rows(BA, N_FILT), _rows(BA, N_FILT)],
        out_shape=[jax.ShapeDtypeStruct((N_ATOMS, N_FILT), F32),
                   jax.ShapeDtypeStruct((N_ATOMS, N_FILT), F32)],
    )(*partials, x, W_fac2out, bfr, W_dense, bdr)

    return (y, v)
